# Initial kernel scaffold; baseline (speedup 1.0000x reference)
import jax
import jax.numpy as jnp
from jax.experimental import pallas as pl


def kernel(features, edge_index, edge_types, W1, b1, W2, b2):
    raise NotImplementedError("write your pallas kernel here")


# speedup vs baseline: 6.2220x; 6.2220x over previous
import functools

import jax
import jax.numpy as jnp
from jax import lax
from jax.experimental import pallas as pl
from jax.experimental.pallas import tpu as pltpu
from jax.experimental.pallas import tpu_sc as plsc

N = 10000
E = 160000
D = 256

NC = 2
NS = 16
NW = NC * NS

NPAD = 10240
ROWS_PER_TILE = NPAD // NS
EP = 163840
EROWS = EP // 128
TRASH = N

_mesh = plsc.VectorSubcoreMesh(
    core_axis_name="c", subcore_axis_name="s", num_cores=NC, num_subcores=NS)


def _fill_zeros(ref, n):
  z = jnp.zeros((16,), jnp.float32)

  @pl.loop(0, n // 16)
  def _(i):
    ref[pl.ds(i * 16, 16)] = z


@functools.partial(
    pl.kernel,
    out_type=(
        jax.ShapeDtypeStruct((NC, NPAD), jnp.float32),
        jax.ShapeDtypeStruct((NC, NPAD), jnp.float32),
    ),
    mesh=_mesh,
    scratch_types=dict(
        sidx=pltpu.VMEM((40, 128), jnp.int32),
        didx=pltpu.VMEM((40, 128), jnp.int32),
        ones=pltpu.VMEM((128,), jnp.float32),
        zer=pltpu.VMEM((ROWS_PER_TILE,), jnp.float32),
        hs=pltpu.VMEM_SHARED((NPAD,), jnp.float32),
        hd=pltpu.VMEM_SHARED((NPAD,), jnp.float32),
    ),
)
def _degrees(src_hbm, dst_hbm, os_hbm, od_hbm, sidx, didx, ones, zer, hs, hd):
  c = lax.axis_index("c")
  s = lax.axis_index("s")
  wid = c * NS + s

  one = jnp.ones((16,), jnp.float32)

  @pl.loop(0, 8)
  def _(i):
    ones[pl.ds(i * 16, 16)] = one

  _fill_zeros(zer, ROWS_PER_TILE)
  pltpu.sync_copy(zer, hs.at[pl.ds(s * ROWS_PER_TILE, ROWS_PER_TILE)])
  pltpu.sync_copy(zer, hd.at[pl.ds(s * ROWS_PER_TILE, ROWS_PER_TILE)])

  pltpu.sync_copy(src_hbm.at[pl.ds(wid * 40, 40)], sidx)
  pltpu.sync_copy(dst_hbm.at[pl.ds(wid * 40, 40)], didx)

  plsc.subcore_barrier()

  @pl.loop(0, 40)
  def _(j):
    pltpu.sync_copy(ones, hs.at[sidx.at[j]], add=True)
    pltpu.sync_copy(ones, hd.at[didx.at[j]], add=True)

  plsc.subcore_barrier()

  sl = pl.ds(s * ROWS_PER_TILE, ROWS_PER_TILE)
  pltpu.sync_copy(hs.at[sl], os_hbm.at[c].at[sl])
  pltpu.sync_copy(hd.at[sl], od_hbm.at[c].at[sl])


@functools.partial(
    pl.kernel,
    out_type=jax.ShapeDtypeStruct((NC, NPAD, 128), jnp.float32),
    mesh=_mesh,
    scratch_types=dict(
        sidx=pltpu.VMEM((40, 128), jnp.int32),
        didx=pltpu.VMEM((40, 128), jnp.int32),
        buf0=pltpu.VMEM((128, 128), jnp.float32),
        buf1=pltpu.VMEM((128, 128), jnp.float32),
        agg=pltpu.VMEM_SHARED((NPAD, 128), jnp.float32),
        sem0=pltpu.SemaphoreType.DMA,
        sem1=pltpu.SemaphoreType.DMA,
    ),
)
def _aggregate(h_hbm, src_hbm, dst_hbm, out_hbm,
               sidx, didx, buf0, buf1, agg, sem0, sem1):
  c = lax.axis_index("c")
  s = lax.axis_index("s")

  @pl.loop(0, 128)
  def _(i):
    z = jnp.zeros((16,), jnp.float32)
    for k in range(8):
      buf0[i, pl.ds(k * 16, 16)] = z

  for k in range(5):
    pltpu.sync_copy(buf0, agg.at[pl.ds(s * ROWS_PER_TILE + k * 128, 128)])

  plsc.subcore_barrier()

  tab = h_hbm.at[c]

  for half in range(2):
    base = s * 80 + half * 40
    pltpu.sync_copy(src_hbm.at[pl.ds(base, 40)], sidx)
    pltpu.sync_copy(dst_hbm.at[pl.ds(base, 40)], didx)

    pltpu.async_copy(tab.at[sidx.at[0]], buf0, sem0)

    @pl.loop(0, 40, step=2)
    def _(j):
      pltpu.make_async_copy(tab.at[sidx.at[j]], buf0, sem0).wait()
      pltpu.async_copy(tab.at[sidx.at[j + 1]], buf1, sem1)
      pltpu.sync_copy(buf0, agg.at[didx.at[j]], add=True)
      pltpu.make_async_copy(tab.at[sidx.at[j + 1]], buf1, sem1).wait()

      @pl.when(j + 2 < 40)
      def _():
        pltpu.async_copy(tab.at[sidx.at[j + 2]], buf0, sem0)

      pltpu.sync_copy(buf1, agg.at[didx.at[j + 1]], add=True)

  plsc.subcore_barrier()

  for k in range(5):
    sl = pl.ds(s * ROWS_PER_TILE + k * 128, 128)
    pltpu.sync_copy(agg.at[sl], out_hbm.at[c].at[sl])


@functools.partial(
    pl.kernel,
    out_type=jax.ShapeDtypeStruct((NC, NPAD), jnp.float32),
    mesh=_mesh,
    scratch_types=dict(
        sidx=pltpu.VMEM((40, 128), jnp.int32),
        didx=pltpu.VMEM((40, 128), jnp.int32),
        val0=pltpu.VMEM((128,), jnp.float32),
        val1=pltpu.VMEM((128,), jnp.float32),
        zer=pltpu.VMEM((ROWS_PER_TILE,), jnp.float32),
        acc=pltpu.VMEM_SHARED((NPAD,), jnp.float32),
        sem0=pltpu.SemaphoreType.DMA,
        sem1=pltpu.SemaphoreType.DMA,
    ),
)
def _scalar_aggregate(w_hbm, src_hbm, dst_hbm, out_hbm,
                      sidx, didx, val0, val1, zer, acc, sem0, sem1):
  c = lax.axis_index("c")
  s = lax.axis_index("s")
  wid = c * NS + s

  _fill_zeros(zer, ROWS_PER_TILE)
  pltpu.sync_copy(zer, acc.at[pl.ds(s * ROWS_PER_TILE, ROWS_PER_TILE)])

  pltpu.sync_copy(src_hbm.at[pl.ds(wid * 40, 40)], sidx)
  pltpu.sync_copy(dst_hbm.at[pl.ds(wid * 40, 40)], didx)

  plsc.subcore_barrier()

  pltpu.async_copy(w_hbm.at[sidx.at[0]], val0, sem0)

  @pl.loop(0, 40, step=2)
  def _(j):
    pltpu.make_async_copy(w_hbm.at[sidx.at[j]], val0, sem0).wait()
    pltpu.async_copy(w_hbm.at[sidx.at[j + 1]], val1, sem1)
    pltpu.sync_copy(val0, acc.at[didx.at[j]], add=True)
    pltpu.make_async_copy(w_hbm.at[sidx.at[j + 1]], val1, sem1).wait()

    @pl.when(j + 2 < 40)
    def _():
      pltpu.async_copy(w_hbm.at[sidx.at[j + 2]], val0, sem0)

    pltpu.sync_copy(val1, acc.at[didx.at[j + 1]], add=True)

  plsc.subcore_barrier()

  sl = pl.ds(s * ROWS_PER_TILE, ROWS_PER_TILE)
  pltpu.sync_copy(acc.at[sl], out_hbm.at[c].at[sl])


TM = 400


def _mm_body(x_ref, w_ref, ns_ref, o_ref):
  y = jnp.dot(x_ref[...], w_ref[...], preferred_element_type=jnp.float32)
  y = y * ns_ref[...]
  o_ref[0] = y[:, :128]
  o_ref[1] = y[:, 128:]


def _epilogue_body(a_ref, nd_ref, ns_ref, w2_ref, b1_ref, o_ref):
  x = jnp.concatenate([a_ref[0], a_ref[1]], axis=1)
  x = jnp.maximum(x * nd_ref[...] + b1_ref[...], 0.0)
  z = jnp.dot(x, w2_ref[...], preferred_element_type=jnp.float32)
  o_ref[...] = z * ns_ref[...]


def kernel(features, edge_index, edge_types, W1, b1, W2, b2):
  del edge_types
  src = edge_index[0]
  dst = edge_index[1]

  pad = EP - E
  src_p = jnp.concatenate([src, jnp.zeros((pad,), jnp.int32)]).reshape(EROWS, 128)
  dst_p = jnp.concatenate([dst, jnp.full((pad,), TRASH, jnp.int32)]).reshape(EROWS, 128)

  hs, hd = _degrees(src_p, dst_p)
  deg_out = hs[0, :N] + hs[1, :N]
  deg_in = hd[0, :N] + hd[1, :N]
  ns = lax.rsqrt(jnp.maximum(deg_out, 1.0))[:, None]
  nd = lax.rsqrt(jnp.maximum(deg_in, 1.0))[:, None]

  h_split = pl.pallas_call(
      _mm_body,
      grid=(N // TM,),
      in_specs=[
          pl.BlockSpec((TM, D), lambda i: (i, 0)),
          pl.BlockSpec((D, D), lambda i: (0, 0)),
          pl.BlockSpec((TM, 1), lambda i: (i, 0)),
      ],
      out_specs=pl.BlockSpec((NC, TM, 128), lambda i: (0, i, 0)),
      out_shape=jax.ShapeDtypeStruct((NC, N, 128), jnp.float32),
  )(features, W1, ns)

  agg = _aggregate(h_split, src_p, dst_p)

  w = pl.pallas_call(
      _epilogue_body,
      grid=(N // TM,),
      in_specs=[
          pl.BlockSpec((NC, TM, 128), lambda i: (0, i, 0)),
          pl.BlockSpec((TM, 1), lambda i: (i, 0)),
          pl.BlockSpec((TM, 1), lambda i: (i, 0)),
          pl.BlockSpec((D, 1), lambda i: (0, 0)),
          pl.BlockSpec((1, D), lambda i: (0, 0)),
      ],
      out_specs=pl.BlockSpec((TM, 1), lambda i: (i, 0)),
      out_shape=jax.ShapeDtypeStruct((N, 1), jnp.float32),
  )(agg, nd, ns, W2, b1.reshape(1, D))

  w_p = jnp.pad(w[:, 0], (0, NPAD - N))
  sacc = _scalar_aggregate(w_p, src_p, dst_p)
  s_sum = sacc[0, :N] + sacc[1, :N]
  return jax.nn.sigmoid(s_sum * nd[:, 0] + b2[0])


# speedup vs baseline: 6.3074x; 1.0137x over previous
import functools

import jax
import jax.numpy as jnp
from jax import lax
from jax.experimental import pallas as pl
from jax.experimental.pallas import tpu as pltpu
from jax.experimental.pallas import tpu_sc as plsc

N = 10000
E = 160000
D = 256

NC = 2
NS = 16
NW = NC * NS

NPAD = 10240
ROWS_PER_TILE = NPAD // NS
EP = 163840
EROWS = EP // 128
TRASH = N

_mesh = plsc.VectorSubcoreMesh(
    core_axis_name="c", subcore_axis_name="s", num_cores=NC, num_subcores=NS)


def _fill_zeros(ref, n):
  z = jnp.zeros((16,), jnp.float32)

  @pl.loop(0, n // 16)
  def _(i):
    ref[pl.ds(i * 16, 16)] = z


@functools.partial(
    pl.kernel,
    out_type=(
        jax.ShapeDtypeStruct((NC, NPAD), jnp.float32),
        jax.ShapeDtypeStruct((NC, NPAD), jnp.float32),
    ),
    mesh=_mesh,
    scratch_types=dict(
        sidx=pltpu.VMEM((40, 128), jnp.int32),
        didx=pltpu.VMEM((40, 128), jnp.int32),
        ones=pltpu.VMEM((128,), jnp.float32),
        zer=pltpu.VMEM((ROWS_PER_TILE,), jnp.float32),
        hs=pltpu.VMEM_SHARED((NPAD,), jnp.float32),
        hd=pltpu.VMEM_SHARED((NPAD,), jnp.float32),
        sem0=pltpu.SemaphoreType.DMA,
        sem1=pltpu.SemaphoreType.DMA,
    ),
)
def _degrees(src_hbm, dst_hbm, os_hbm, od_hbm,
             sidx, didx, ones, zer, hs, hd, sem0, sem1):
  c = lax.axis_index("c")
  s = lax.axis_index("s")
  wid = c * NS + s

  one = jnp.ones((16,), jnp.float32)

  @pl.loop(0, 8)
  def _(i):
    ones[pl.ds(i * 16, 16)] = one

  _fill_zeros(zer, ROWS_PER_TILE)
  pltpu.sync_copy(zer, hs.at[pl.ds(s * ROWS_PER_TILE, ROWS_PER_TILE)])
  pltpu.sync_copy(zer, hd.at[pl.ds(s * ROWS_PER_TILE, ROWS_PER_TILE)])

  pltpu.sync_copy(src_hbm.at[pl.ds(wid * 40, 40)], sidx)
  pltpu.sync_copy(dst_hbm.at[pl.ds(wid * 40, 40)], didx)

  plsc.subcore_barrier()

  @pl.loop(0, 40, step=8)
  def _(j):
    for b in range(8):
      pltpu.async_copy(ones, hs.at[sidx.at[j + b]], sem0, add=True)
      pltpu.async_copy(ones, hd.at[didx.at[j + b]], sem1, add=True)
    for b in range(8):
      pltpu.make_async_copy(ones, hs.at[sidx.at[j + b]], sem0).wait()
      pltpu.make_async_copy(ones, hd.at[didx.at[j + b]], sem1).wait()

  plsc.subcore_barrier()

  sl = pl.ds(s * ROWS_PER_TILE, ROWS_PER_TILE)
  pltpu.sync_copy(hs.at[sl], os_hbm.at[c].at[sl])
  pltpu.sync_copy(hd.at[sl], od_hbm.at[c].at[sl])


@functools.partial(
    pl.kernel,
    out_type=jax.ShapeDtypeStruct((NC, NPAD, 128), jnp.float32),
    mesh=_mesh,
    scratch_types=dict(
        sidx=pltpu.VMEM((40, 128), jnp.int32),
        didx=pltpu.VMEM((40, 128), jnp.int32),
        buf0=pltpu.VMEM((128, 128), jnp.float32),
        buf1=pltpu.VMEM((128, 128), jnp.float32),
        agg=pltpu.VMEM_SHARED((NPAD, 128), jnp.float32),
        gs0=pltpu.SemaphoreType.DMA,
        gs1=pltpu.SemaphoreType.DMA,
        ss0=pltpu.SemaphoreType.DMA,
        ss1=pltpu.SemaphoreType.DMA,
    ),
)
def _aggregate(h_hbm, src_hbm, dst_hbm, out_hbm,
               sidx, didx, buf0, buf1, agg, gs0, gs1, ss0, ss1):
  c = lax.axis_index("c")
  s = lax.axis_index("s")

  @pl.loop(0, 128)
  def _(i):
    z = jnp.zeros((16,), jnp.float32)
    for k in range(8):
      buf0[i, pl.ds(k * 16, 16)] = z

  for k in range(5):
    pltpu.sync_copy(buf0, agg.at[pl.ds(s * ROWS_PER_TILE + k * 128, 128)])

  plsc.subcore_barrier()

  tab = h_hbm.at[c]

  for half in range(2):
    base = s * 80 + half * 40
    pltpu.sync_copy(src_hbm.at[pl.ds(base, 40)], sidx)
    pltpu.sync_copy(dst_hbm.at[pl.ds(base, 40)], didx)

    pltpu.async_copy(tab.at[sidx.at[0]], buf0, gs0)
    pltpu.async_copy(tab.at[sidx.at[1]], buf1, gs1)

    @pl.loop(0, 40, step=2)
    def _(j):
      pltpu.make_async_copy(tab.at[sidx.at[j]], buf0, gs0).wait()
      pltpu.async_copy(buf0, agg.at[didx.at[j]], ss0, add=True)
      pltpu.make_async_copy(tab.at[sidx.at[j + 1]], buf1, gs1).wait()
      pltpu.async_copy(buf1, agg.at[didx.at[j + 1]], ss1, add=True)

      @pl.when(j + 2 < 40)
      def _():
        pltpu.make_async_copy(buf0, agg.at[didx.at[j]], ss0).wait()
        pltpu.async_copy(tab.at[sidx.at[j + 2]], buf0, gs0)
        pltpu.make_async_copy(buf1, agg.at[didx.at[j + 1]], ss1).wait()
        pltpu.async_copy(tab.at[sidx.at[j + 3]], buf1, gs1)

    pltpu.make_async_copy(buf0, agg.at[didx.at[38]], ss0).wait()
    pltpu.make_async_copy(buf1, agg.at[didx.at[39]], ss1).wait()

  plsc.subcore_barrier()

  for k in range(5):
    sl = pl.ds(s * ROWS_PER_TILE + k * 128, 128)
    pltpu.sync_copy(agg.at[sl], out_hbm.at[c].at[sl])


@functools.partial(
    pl.kernel,
    out_type=jax.ShapeDtypeStruct((NC, NPAD), jnp.float32),
    mesh=_mesh,
    scratch_types=dict(
        sidx=pltpu.VMEM((40, 128), jnp.int32),
        didx=pltpu.VMEM((40, 128), jnp.int32),
        val0=pltpu.VMEM((128,), jnp.float32),
        val1=pltpu.VMEM((128,), jnp.float32),
        val2=pltpu.VMEM((128,), jnp.float32),
        val3=pltpu.VMEM((128,), jnp.float32),
        zer=pltpu.VMEM((ROWS_PER_TILE,), jnp.float32),
        acc=pltpu.VMEM_SHARED((NPAD,), jnp.float32),
        gs0=pltpu.SemaphoreType.DMA,
        gs1=pltpu.SemaphoreType.DMA,
        gs2=pltpu.SemaphoreType.DMA,
        gs3=pltpu.SemaphoreType.DMA,
        ss0=pltpu.SemaphoreType.DMA,
        ss1=pltpu.SemaphoreType.DMA,
        ss2=pltpu.SemaphoreType.DMA,
        ss3=pltpu.SemaphoreType.DMA,
    ),
)
def _scalar_aggregate(w_hbm, src_hbm, dst_hbm, out_hbm, sidx, didx,
                      val0, val1, val2, val3, zer, acc,
                      gs0, gs1, gs2, gs3, ss0, ss1, ss2, ss3):
  c = lax.axis_index("c")
  s = lax.axis_index("s")
  wid = c * NS + s

  _fill_zeros(zer, ROWS_PER_TILE)
  pltpu.sync_copy(zer, acc.at[pl.ds(s * ROWS_PER_TILE, ROWS_PER_TILE)])

  pltpu.sync_copy(src_hbm.at[pl.ds(wid * 40, 40)], sidx)
  pltpu.sync_copy(dst_hbm.at[pl.ds(wid * 40, 40)], didx)

  plsc.subcore_barrier()

  vals = (val0, val1, val2, val3)
  gsems = (gs0, gs1, gs2, gs3)
  ssems = (ss0, ss1, ss2, ss3)

  for b in range(4):
    pltpu.async_copy(w_hbm.at[sidx.at[b]], vals[b], gsems[b])

  @pl.loop(0, 40, step=4)
  def _(j):
    for b in range(4):
      pltpu.make_async_copy(w_hbm.at[sidx.at[j + b]], vals[b], gsems[b]).wait()
      pltpu.async_copy(vals[b], acc.at[didx.at[j + b]], ssems[b], add=True)

    @pl.when(j + 4 < 40)
    def _():
      for b in range(4):
        pltpu.make_async_copy(vals[b], acc.at[didx.at[j + b]], ssems[b]).wait()
        pltpu.async_copy(w_hbm.at[sidx.at[j + 4 + b]], vals[b], gsems[b])

  for b in range(4):
    pltpu.make_async_copy(vals[b], acc.at[didx.at[36 + b]], ssems[b]).wait()

  plsc.subcore_barrier()

  sl = pl.ds(s * ROWS_PER_TILE, ROWS_PER_TILE)
  pltpu.sync_copy(acc.at[sl], out_hbm.at[c].at[sl])


TM = 400


def _mm_body(x_ref, w_ref, ns_ref, o_ref):
  y = jnp.dot(x_ref[...], w_ref[...], preferred_element_type=jnp.float32)
  y = y * ns_ref[...]
  o_ref[0] = y[:, :128]
  o_ref[1] = y[:, 128:]


def _epilogue_body(a_ref, nd_ref, ns_ref, w2_ref, b1_ref, o_ref):
  x = jnp.concatenate([a_ref[0], a_ref[1]], axis=1)
  x = jnp.maximum(x * nd_ref[...] + b1_ref[...], 0.0)
  z = jnp.dot(x, w2_ref[...], preferred_element_type=jnp.float32)
  o_ref[...] = z * ns_ref[...]


def kernel(features, edge_index, edge_types, W1, b1, W2, b2):
  del edge_types
  src = edge_index[0]
  dst = edge_index[1]

  pad = EP - E
  src_p = jnp.concatenate([src, jnp.zeros((pad,), jnp.int32)]).reshape(EROWS, 128)
  dst_p = jnp.concatenate([dst, jnp.full((pad,), TRASH, jnp.int32)]).reshape(EROWS, 128)

  hs, hd = _degrees(src_p, dst_p)
  deg_out = hs[0, :N] + hs[1, :N]
  deg_in = hd[0, :N] + hd[1, :N]
  ns = lax.rsqrt(jnp.maximum(deg_out, 1.0))[:, None]
  nd = lax.rsqrt(jnp.maximum(deg_in, 1.0))[:, None]

  h_split = pl.pallas_call(
      _mm_body,
      grid=(N // TM,),
      in_specs=[
          pl.BlockSpec((TM, D), lambda i: (i, 0)),
          pl.BlockSpec((D, D), lambda i: (0, 0)),
          pl.BlockSpec((TM, 1), lambda i: (i, 0)),
      ],
      out_specs=pl.BlockSpec((NC, TM, 128), lambda i: (0, i, 0)),
      out_shape=jax.ShapeDtypeStruct((NC, N, 128), jnp.float32),
  )(features, W1, ns)

  agg = _aggregate(h_split, src_p, dst_p)

  w = pl.pallas_call(
      _epilogue_body,
      grid=(N // TM,),
      in_specs=[
          pl.BlockSpec((NC, TM, 128), lambda i: (0, i, 0)),
          pl.BlockSpec((TM, 1), lambda i: (i, 0)),
          pl.BlockSpec((TM, 1), lambda i: (i, 0)),
          pl.BlockSpec((D, 1), lambda i: (0, 0)),
          pl.BlockSpec((1, D), lambda i: (0, 0)),
      ],
      out_specs=pl.BlockSpec((TM, 1), lambda i: (i, 0)),
      out_shape=jax.ShapeDtypeStruct((N, 1), jnp.float32),
  )(agg, nd, ns, W2, b1.reshape(1, D))

  w_p = jnp.pad(w[:, 0], (0, NPAD - N))
  sacc = _scalar_aggregate(w_p, src_p, dst_p)
  s_sum = sacc[0, :N] + sacc[1, :N]
  return jax.nn.sigmoid(s_sum * nd[:, 0] + b2[0])


# speedup vs baseline: 6.8085x; 1.0795x over previous
import functools

import jax
import jax.numpy as jnp
from jax import lax
from jax.experimental import pallas as pl
from jax.experimental.pallas import tpu as pltpu
from jax.experimental.pallas import tpu_sc as plsc

N = 10000
E = 160000
D = 256

NC = 2
NS = 16
NW = NC * NS

NPAD = 10240
ROWS_PER_TILE = NPAD // NS
EP = 163840
EROWS = EP // 128
TRASH = N

_mesh = plsc.VectorSubcoreMesh(
    core_axis_name="c", subcore_axis_name="s", num_cores=NC, num_subcores=NS)


def _fill_zeros(ref, n):
  z = jnp.zeros((16,), jnp.float32)

  @pl.loop(0, n // 16)
  def _(i):
    ref[pl.ds(i * 16, 16)] = z


@functools.partial(
    pl.kernel,
    out_type=(
        jax.ShapeDtypeStruct((NC, NPAD), jnp.float32),
        jax.ShapeDtypeStruct((NC, NPAD), jnp.float32),
    ),
    mesh=_mesh,
    scratch_types=dict(
        sidx=pltpu.VMEM((40, 128), jnp.int32),
        didx=pltpu.VMEM((40, 128), jnp.int32),
        ones=pltpu.VMEM((128,), jnp.float32),
        zer=pltpu.VMEM((ROWS_PER_TILE,), jnp.float32),
        hs=pltpu.VMEM_SHARED((NPAD,), jnp.float32),
        hd=pltpu.VMEM_SHARED((NPAD,), jnp.float32),
        sem0=pltpu.SemaphoreType.DMA,
        sem1=pltpu.SemaphoreType.DMA,
    ),
)
def _degrees(src_hbm, dst_hbm, os_hbm, od_hbm,
             sidx, didx, ones, zer, hs, hd, sem0, sem1):
  c = lax.axis_index("c")
  s = lax.axis_index("s")
  wid = c * NS + s

  one = jnp.ones((16,), jnp.float32)

  @pl.loop(0, 8)
  def _(i):
    ones[pl.ds(i * 16, 16)] = one

  _fill_zeros(zer, ROWS_PER_TILE)
  pltpu.sync_copy(zer, hs.at[pl.ds(s * ROWS_PER_TILE, ROWS_PER_TILE)])
  pltpu.sync_copy(zer, hd.at[pl.ds(s * ROWS_PER_TILE, ROWS_PER_TILE)])

  pltpu.sync_copy(src_hbm.at[pl.ds(wid * 40, 40)], sidx)
  pltpu.sync_copy(dst_hbm.at[pl.ds(wid * 40, 40)], didx)

  plsc.subcore_barrier()

  @pl.loop(0, 40, step=8)
  def _(j):
    for b in range(8):
      pltpu.async_copy(ones, hs.at[sidx.at[j + b]], sem0, add=True)
      pltpu.async_copy(ones, hd.at[didx.at[j + b]], sem1, add=True)
    for b in range(8):
      pltpu.make_async_copy(ones, hs.at[sidx.at[j + b]], sem0).wait()
      pltpu.make_async_copy(ones, hd.at[didx.at[j + b]], sem1).wait()

  plsc.subcore_barrier()

  sl = pl.ds(s * ROWS_PER_TILE, ROWS_PER_TILE)
  pltpu.sync_copy(hs.at[sl], os_hbm.at[c].at[sl])
  pltpu.sync_copy(hd.at[sl], od_hbm.at[c].at[sl])


@functools.partial(
    pl.kernel,
    out_type=jax.ShapeDtypeStruct((NC, NPAD), jnp.float32),
    mesh=_mesh,
    scratch_types=dict(
        sidx=pltpu.VMEM((40, 128), jnp.int32),
        didx=pltpu.VMEM((40, 128), jnp.int32),
        buf0=pltpu.VMEM((128, 128), jnp.float32),
        buf1=pltpu.VMEM((128, 128), jnp.float32),
        ndv=pltpu.VMEM((ROWS_PER_TILE + 16,), jnp.float32),
        nsv=pltpu.VMEM((ROWS_PER_TILE,), jnp.float32),
        zv=pltpu.VMEM((ROWS_PER_TILE,), jnp.float32),
        w2v=pltpu.VMEM((128,), jnp.float32),
        b1v=pltpu.VMEM((128,), jnp.float32),
        val0=pltpu.VMEM((128,), jnp.float32),
        val1=pltpu.VMEM((128,), jnp.float32),
        val2=pltpu.VMEM((128,), jnp.float32),
        val3=pltpu.VMEM((128,), jnp.float32),
        idx0=pltpu.VMEM((128,), jnp.int32),
        idx1=pltpu.VMEM((128,), jnp.int32),
        agg=pltpu.VMEM_SHARED((NPAD, 128), jnp.float32),
        acc=pltpu.VMEM_SHARED((NPAD,), jnp.float32),
        zsh=pltpu.VMEM_SHARED((NPAD,), jnp.float32),
        wsh=pltpu.VMEM_SHARED((NPAD,), jnp.float32),
        gs0=pltpu.SemaphoreType.DMA,
        gs1=pltpu.SemaphoreType.DMA,
        gs2=pltpu.SemaphoreType.DMA,
        gs3=pltpu.SemaphoreType.DMA,
        ss0=pltpu.SemaphoreType.DMA,
        ss1=pltpu.SemaphoreType.DMA,
        ss2=pltpu.SemaphoreType.DMA,
        ss3=pltpu.SemaphoreType.DMA,
    ),
)
def _aggregate(h_hbm, src_hbm, dst_hbm, nd_hbm, ns_hbm, w2_hbm, b1_hbm,
               out_hbm,
               sidx, didx, buf0, buf1, ndv, nsv, zv, w2v, b1v,
               val0, val1, val2, val3, idx0, idx1, agg, acc, zsh, wsh,
               gs0, gs1, gs2, gs3, ss0, ss1, ss2, ss3):
  c = lax.axis_index("c")
  s = lax.axis_index("s")
  myrows = pl.ds(s * ROWS_PER_TILE, ROWS_PER_TILE)

  @pl.loop(0, 128)
  def _(i):
    z = jnp.zeros((16,), jnp.float32)
    for k in range(8):
      buf0[i, pl.ds(k * 16, 16)] = z

  for k in range(5):
    pltpu.sync_copy(buf0, agg.at[pl.ds(s * ROWS_PER_TILE + k * 128, 128)])
  _fill_zeros(zv, ROWS_PER_TILE)
  pltpu.sync_copy(zv, acc.at[myrows])
  pltpu.sync_copy(zv, zsh.at[myrows])

  plsc.subcore_barrier()

  tab = h_hbm.at[c]

  for half in range(2):
    base = s * 80 + half * 40
    pltpu.sync_copy(src_hbm.at[pl.ds(base, 40)], sidx)
    pltpu.sync_copy(dst_hbm.at[pl.ds(base, 40)], didx)

    pltpu.async_copy(tab.at[sidx.at[0]], buf0, gs0)
    pltpu.async_copy(tab.at[sidx.at[1]], buf1, gs1)

    @pl.loop(0, 40, step=2)
    def _(j):
      pltpu.make_async_copy(tab.at[sidx.at[j]], buf0, gs0).wait()
      pltpu.async_copy(buf0, agg.at[didx.at[j]], ss0, add=True)
      pltpu.make_async_copy(tab.at[sidx.at[j + 1]], buf1, gs1).wait()
      pltpu.async_copy(buf1, agg.at[didx.at[j + 1]], ss1, add=True)

      @pl.when(j + 2 < 40)
      def _():
        pltpu.make_async_copy(buf0, agg.at[didx.at[j]], ss0).wait()
        pltpu.async_copy(tab.at[sidx.at[j + 2]], buf0, gs0)
        pltpu.make_async_copy(buf1, agg.at[didx.at[j + 1]], ss1).wait()
        pltpu.async_copy(tab.at[sidx.at[j + 3]], buf1, gs1)

    pltpu.make_async_copy(buf0, agg.at[didx.at[38]], ss0).wait()
    pltpu.make_async_copy(buf1, agg.at[didx.at[39]], ss1).wait()

  pltpu.sync_copy(nd_hbm.at[myrows], ndv.at[pl.ds(0, ROWS_PER_TILE)])
  pltpu.sync_copy(ns_hbm.at[myrows], nsv)
  pltpu.sync_copy(w2_hbm.at[c], w2v)
  pltpu.sync_copy(b1_hbm.at[c], b1v)

  plsc.subcore_barrier()

  vals2 = (val0, val1)
  idxs2 = (idx0, idx1)
  sems2 = (ss2, ss3)

  @pl.loop(0, 5)
  def _(k5):
    pltpu.sync_copy(agg.at[pl.ds(s * ROWS_PER_TILE + k5 * 128, 128)], buf1)

    @pl.loop(0, 16, step=2)
    def _(g2):
      for slot in range(2):
        g = g2 + slot
        nd16 = ndv[pl.ds(k5 * 128 + g * 8, 16)]

        @pl.when(jnp.logical_or(k5 > 0, g2 > 0))
        def _():
          pltpu.make_async_copy(
              vals2[slot], zsh.at[idxs2[slot]], sems2[slot]).wait()

        for i in range(8):
          d = nd16[i]
          a16 = jnp.zeros((16,), jnp.float32)
          for k in range(8):
            x = buf1[g * 8 + i, pl.ds(k * 16, 16)] * d + b1v[pl.ds(k * 16, 16)]
            a16 = a16 + jnp.maximum(x, 0.0) * w2v[pl.ds(k * 16, 16)]
          vals2[slot][pl.ds(i * 16, 16)] = a16
          idxs2[slot][pl.ds(i * 16, 16)] = jnp.full(
              (16,), s * ROWS_PER_TILE + k5 * 128 + g * 8 + i, jnp.int32)
        pltpu.async_copy(vals2[slot], zsh.at[idxs2[slot]], sems2[slot],
                         add=True)

  for slot in range(2):
    pltpu.make_async_copy(vals2[slot], zsh.at[idxs2[slot]], sems2[slot]).wait()

  plsc.subcore_barrier()

  pltpu.sync_copy(zsh.at[myrows], zv)

  @pl.loop(0, ROWS_PER_TILE // 16)
  def _(i):
    zv[pl.ds(i * 16, 16)] = zv[pl.ds(i * 16, 16)] * nsv[pl.ds(i * 16, 16)]

  pltpu.sync_copy(zv, wsh.at[myrows])

  plsc.subcore_barrier()

  wtab = wsh
  vals = (val0, val1, val2, val3)
  gsems = (gs0, gs1, gs2, gs3)
  ssems = (ss0, ss1, ss2, ss3)

  for half in range(2):
    base = s * 80 + half * 40
    pltpu.sync_copy(src_hbm.at[pl.ds(base, 40)], sidx)
    pltpu.sync_copy(dst_hbm.at[pl.ds(base, 40)], didx)

    for b in range(4):
      pltpu.async_copy(wtab.at[sidx.at[b]], vals[b], gsems[b])

    @pl.loop(0, 40, step=4)
    def _(j):
      for b in range(4):
        pltpu.make_async_copy(wtab.at[sidx.at[j + b]], vals[b], gsems[b]).wait()
        pltpu.async_copy(vals[b], acc.at[didx.at[j + b]], ssems[b], add=True)

      @pl.when(j + 4 < 40)
      def _():
        for b in range(4):
          pltpu.make_async_copy(vals[b], acc.at[didx.at[j + b]], ssems[b]).wait()
          pltpu.async_copy(wtab.at[sidx.at[j + 4 + b]], vals[b], gsems[b])

    for b in range(4):
      pltpu.make_async_copy(vals[b], acc.at[didx.at[36 + b]], ssems[b]).wait()

  plsc.subcore_barrier()

  pltpu.sync_copy(acc.at[myrows], out_hbm.at[c].at[myrows])


TM = 400


def _mm_body(x_ref, w_ref, ns_ref, o_ref):
  y = jnp.dot(x_ref[...], w_ref[...], preferred_element_type=jnp.float32)
  y = y * ns_ref[...]
  o_ref[0] = y[:, :128]
  o_ref[1] = y[:, 128:]


def kernel(features, edge_index, edge_types, W1, b1, W2, b2):
  del edge_types
  src = edge_index[0]
  dst = edge_index[1]

  pad = EP - E
  src_p = jnp.concatenate([src, jnp.zeros((pad,), jnp.int32)]).reshape(EROWS, 128)
  dst_p = jnp.concatenate([dst, jnp.full((pad,), TRASH, jnp.int32)]).reshape(EROWS, 128)

  hs, hd = _degrees(src_p, dst_p)
  deg_out = hs[0, :N] + hs[1, :N]
  deg_in = hd[0, :N] + hd[1, :N]
  ns = lax.rsqrt(jnp.maximum(deg_out, 1.0))[:, None]
  nd = lax.rsqrt(jnp.maximum(deg_in, 1.0))[:, None]

  h_split = pl.pallas_call(
      _mm_body,
      grid=(N // TM,),
      in_specs=[
          pl.BlockSpec((TM, D), lambda i: (i, 0)),
          pl.BlockSpec((D, D), lambda i: (0, 0)),
          pl.BlockSpec((TM, 1), lambda i: (i, 0)),
      ],
      out_specs=pl.BlockSpec((NC, TM, 128), lambda i: (0, i, 0)),
      out_shape=jax.ShapeDtypeStruct((NC, N, 128), jnp.float32),
  )(features, W1, ns)

  nd_p = jnp.pad(nd[:, 0], (0, NPAD - N))
  ns_p = jnp.pad(ns[:, 0], (0, NPAD - N))
  sacc = _aggregate(h_split, src_p, dst_p, nd_p, ns_p,
                    W2[:, 0].reshape(NC, 128), b1.reshape(NC, 128))

  s_sum = sacc[0, :N] + sacc[1, :N]
  return jax.nn.sigmoid(s_sum * nd[:, 0] + b2[0])
